# baseline (speedup 1.0000x reference)
import functools

import jax
import jax.numpy as jnp
from jax import lax
from jax.experimental import pallas as pl
from jax.experimental.pallas import tpu as pltpu
from jax.experimental.pallas import tpu_sc as plsc

N = 10000
NP = 10240
D = 128
DH = D // 2
E = 320000
EP = 327680
NW = 32
CH = 128
NCH = EP // (NW * CH)
RPT = NP // 16
_U = 8


def _mesh():
    return plsc.VectorSubcoreMesh(core_axis_name="c", subcore_axis_name="s")


@functools.partial(
    pl.kernel,
    mesh=_mesh(),
    out_type=jax.ShapeDtypeStruct((2, NP), jnp.float32),
    scratch_types=[
        pltpu.VMEM((NCH, CH), jnp.int32),
        pltpu.VMEM((CH,), jnp.float32),
        pltpu.VMEM((RPT,), jnp.float32),
        pltpu.VMEM_SHARED((NP,), jnp.float32),
    ],
)
def _sc_hist(col3, out, colbuf, ones_v, hbuf, hist):
    cid = lax.axis_index("c")
    sid = lax.axis_index("s")
    w = cid * 16 + sid
    for l in range(CH // 16):
        ones_v[pl.ds(l * 16, 16)] = jnp.ones((16,), jnp.float32)
    for l in range(RPT // 16):
        hbuf[pl.ds(l * 16, 16)] = jnp.zeros((16,), jnp.float32)
    pltpu.sync_copy(hbuf, hist.at[pl.ds(sid * RPT, RPT)])
    pltpu.sync_copy(col3.at[w], colbuf)
    plsc.subcore_barrier()

    def body(j, carry):
        pltpu.sync_copy(ones_v, hist.at[colbuf.at[j]], add=True)
        return carry

    lax.fori_loop(0, NCH, body, 0)
    plsc.subcore_barrier()
    pltpu.sync_copy(hist.at[pl.ds(sid * RPT, RPT)], hbuf)
    pltpu.sync_copy(hbuf, out.at[cid, pl.ds(sid * RPT, RPT)])


@functools.partial(
    pl.kernel,
    mesh=_mesh(),
    out_type=jax.ShapeDtypeStruct((2, NP, DH), jnp.float32),
    compiler_params=pltpu.CompilerParams(use_tc_tiling_on_sc=False),
    scratch_types=[
        pltpu.VMEM((NCH, CH), jnp.int32),
        pltpu.VMEM((NCH, CH), jnp.int32),
        pltpu.VMEM((CH, DH), jnp.float32),
        pltpu.VMEM((CH, DH), jnp.float32),
        pltpu.VMEM((64, DH), jnp.float32),
        pltpu.VMEM_SHARED((NP, DH), jnp.float32),
        pltpu.SemaphoreType.DMA,
        pltpu.SemaphoreType.DMA,
    ],
)
def _sc_scatter(ht, row3, col3, out, rowbuf, colbuf, rv0, rv1, zbuf, acc,
                sem0, sem1):
    cid = lax.axis_index("c")
    sid = lax.axis_index("s")
    myht = ht.at[cid]
    for i in range(64):
        for l in range(DH // 16):
            zbuf[i, pl.ds(l * 16, 16)] = jnp.zeros((16,), jnp.float32)
    base = sid * RPT
    for t in range(RPT // 64):
        pltpu.sync_copy(zbuf, acc.at[pl.ds(base + t * 64, 64)])

    for p in range(2):
        g = sid * 2 + p
        pltpu.sync_copy(row3.at[g], rowbuf)
        pltpu.sync_copy(col3.at[g], colbuf)
        if p == 0:
            plsc.subcore_barrier()

        def body(i, carry):
            prev = None
            for u in range(_U):
                j = i * _U + u
                rv = rv0 if u % 2 == 0 else rv1
                pltpu.async_copy(myht.at[rowbuf.at[j]], rv, sem0).wait()
                if prev is not None:
                    prev.wait()
                prev = pltpu.async_copy(rv, acc.at[colbuf.at[j]], sem1,
                                        add=True)
            prev.wait()
            return carry

        lax.fori_loop(0, NCH // _U, body, 0)

    plsc.subcore_barrier()
    pltpu.sync_copy(acc.at[pl.ds(base, RPT)], out.at[cid].at[pl.ds(base, RPT)])


_R = 1280
_GRID = NP // _R


def _row_spec():
    return pl.BlockSpec((_R, D), lambda i: (i, 0))


def _half_spec():
    return pl.BlockSpec((2, _R, DH), lambda i: (0, i, 0))


def _dis_spec():
    return pl.BlockSpec((_R, 1), lambda i: (i, 0))


def _full_spec(r, c):
    return pl.BlockSpec((r, c), lambda i: (0, 0))


def _store_halves(h3_ref, h):
    h3_ref[0] = h[:, 0:DH]
    h3_ref[1] = h[:, DH:D]


def _load_halves(h3_ref):
    return jnp.concatenate([h3_ref[0], h3_ref[1]], axis=1)


def _tc_first_body(x_ref, w_ref, dis_ref, ht_ref):
    _store_halves(ht_ref, dis_ref[...] * jnp.dot(
        x_ref[...], w_ref[...], preferred_element_type=jnp.float32))


def _tc_first(xp, W0, dis):
    return pl.pallas_call(
        _tc_first_body,
        grid=(_GRID,),
        in_specs=[_row_spec(), _full_spec(D, D), _dis_spec()],
        out_specs=_half_spec(),
        out_shape=jax.ShapeDtypeStruct((2, NP, DH), jnp.float32),
    )(xp, W0, dis)


def _tc_mid_body(s_ref, ht_ref, dis_ref, b_ref, w_ref, h_ref, htn_ref):
    pre = dis_ref[...] * (_load_halves(s_ref) + _load_halves(ht_ref)) \
        + b_ref[...]
    h = jnp.maximum(pre, 0.0)
    h_ref[...] = h
    _store_halves(htn_ref, dis_ref[...] * jnp.dot(
        h, w_ref[...], preferred_element_type=jnp.float32))


def _tc_mid(s, ht, dis, b, Wn):
    return pl.pallas_call(
        _tc_mid_body,
        grid=(_GRID,),
        in_specs=[_half_spec(), _half_spec(), _dis_spec(),
                  _full_spec(1, D), _full_spec(D, D)],
        out_specs=(_row_spec(), _half_spec()),
        out_shape=(jax.ShapeDtypeStruct((NP, D), jnp.float32),
                   jax.ShapeDtypeStruct((2, NP, DH), jnp.float32)),
    )(s, ht, dis, b, Wn)


def _tc_last_body(s_ref, ht_ref, dis_ref, b_ref, h0_ref, h1_ref,
                  wb_ref, out_ref):
    h2 = dis_ref[...] * (_load_halves(s_ref) + _load_halves(ht_ref)) \
        + b_ref[...]
    out_ref[...] = (wb_ref[0:1, :] * h0_ref[...] + wb_ref[1:2, :] * h1_ref[...]
                    + wb_ref[2:3, :] * h2)


def _tc_last(s, ht, dis, b, h0, h1, wb):
    return pl.pallas_call(
        _tc_last_body,
        grid=(_GRID,),
        in_specs=[_half_spec(), _half_spec(), _dis_spec(),
                  _full_spec(1, D), _row_spec(), _row_spec(), _full_spec(3, D)],
        out_specs=_row_spec(),
        out_shape=jax.ShapeDtypeStruct((NP, D), jnp.float32),
    )(s, ht, dis, b, h0, h1, wb)


def kernel(x, edge_index, W0, b0, W1, b1, W2, b2, jk_w):
    pad = EP - E
    padidx = jnp.int32(N) + (jnp.arange(pad, dtype=jnp.int32) % (NP - N))
    row3 = jnp.concatenate([edge_index[0], padidx]).reshape(NW, NCH, CH)
    col3 = jnp.concatenate([edge_index[1], padidx]).reshape(NW, NCH, CH)

    counts = _sc_hist(col3)
    cnt = counts[0] + counts[1]
    valid = jnp.arange(NP) < N
    dis = jnp.where(valid, lax.rsqrt(cnt + 1.0), 0.0).astype(jnp.float32)
    dis = dis.reshape(NP, 1)

    xp = jnp.concatenate([x, jnp.zeros((NP - N, D), jnp.float32)], axis=0)

    ht0 = _tc_first(xp, W0, dis)
    s0 = _sc_scatter(ht0, row3, col3)
    h0, ht1 = _tc_mid(s0, ht0, dis, b0.reshape(1, D), W1)
    s1 = _sc_scatter(ht1, row3, col3)
    h1, ht2 = _tc_mid(s1, ht1, dis, b1.reshape(1, D), W2)
    s2 = _sc_scatter(ht2, row3, col3)

    w = jax.nn.softmax(jk_w)
    wb = jnp.broadcast_to(w.reshape(3, 1), (3, D)).astype(jnp.float32)
    out = _tc_last(s2, ht2, dis, b2.reshape(1, D), h0, h1, wb)
    return out[:N]


# speedup vs baseline: 1.0701x; 1.0701x over previous
import functools

import jax
import jax.numpy as jnp
from jax import lax
from jax.experimental import pallas as pl
from jax.experimental.pallas import tpu as pltpu
from jax.experimental.pallas import tpu_sc as plsc

N = 10000
NP = 10240
D = 128
E = 320000
EP = 327680
NW = 32
CH = 128
NCH = EP // (NW * CH)
RPT = NP // 16
_U = 8


def _mesh():
    return plsc.VectorSubcoreMesh(core_axis_name="c", subcore_axis_name="s")


@functools.partial(
    pl.kernel,
    mesh=_mesh(),
    out_type=jax.ShapeDtypeStruct((2, NP), jnp.float32),
    scratch_types=[
        pltpu.VMEM((NCH, CH), jnp.int32),
        pltpu.VMEM((CH,), jnp.float32),
        pltpu.VMEM((RPT,), jnp.float32),
        pltpu.VMEM_SHARED((NP,), jnp.float32),
    ],
)
def _sc_hist(col3, out, colbuf, ones_v, hbuf, hist):
    cid = lax.axis_index("c")
    sid = lax.axis_index("s")
    w = cid * 16 + sid
    for l in range(CH // 16):
        ones_v[pl.ds(l * 16, 16)] = jnp.ones((16,), jnp.float32)
    for l in range(RPT // 16):
        hbuf[pl.ds(l * 16, 16)] = jnp.zeros((16,), jnp.float32)
    pltpu.sync_copy(hbuf, hist.at[pl.ds(sid * RPT, RPT)])
    pltpu.sync_copy(col3.at[w], colbuf)
    plsc.subcore_barrier()

    def body(j, carry):
        pltpu.sync_copy(ones_v, hist.at[colbuf.at[j]], add=True)
        return carry

    lax.fori_loop(0, NCH, body, 0)
    plsc.subcore_barrier()
    pltpu.sync_copy(hist.at[pl.ds(sid * RPT, RPT)], hbuf)
    pltpu.sync_copy(hbuf, out.at[cid, pl.ds(sid * RPT, RPT)])


@functools.partial(
    pl.kernel,
    mesh=_mesh(),
    out_type=jax.ShapeDtypeStruct((2, NP, D), jnp.float32),
    scratch_types=[
        pltpu.VMEM((NCH, CH), jnp.int32),
        pltpu.VMEM((NCH, CH), jnp.int32),
        pltpu.VMEM((CH, D), jnp.float32),
        pltpu.VMEM((CH, D), jnp.float32),
        pltpu.VMEM((64, D), jnp.float32),
        pltpu.VMEM_SHARED((NP, D), jnp.float32),
        pltpu.SemaphoreType.DMA,
        pltpu.SemaphoreType.DMA,
    ],
)
def _sc_scatter(ht, row3, col3, out, rowbuf, colbuf, rv0, rv1, zbuf, acc,
                sem0, sem1):
    cid = lax.axis_index("c")
    sid = lax.axis_index("s")
    w = cid * 16 + sid
    for i in range(64):
        for l in range(D // 16):
            zbuf[i, pl.ds(l * 16, 16)] = jnp.zeros((16,), jnp.float32)
    base = sid * RPT
    for t in range(RPT // 64):
        pltpu.sync_copy(zbuf, acc.at[pl.ds(base + t * 64, 64)])
    pltpu.sync_copy(row3.at[w], rowbuf)
    pltpu.sync_copy(col3.at[w], colbuf)
    plsc.subcore_barrier()

    def body(j, carry):
        pltpu.async_copy(ht.at[rowbuf.at[j]], rv0, sem0).wait()
        pltpu.async_copy(rv0, acc.at[colbuf.at[j]], sem1, add=True).wait()
        return carry

    lax.fori_loop(0, NCH, body, 0)
    plsc.subcore_barrier()
    pltpu.sync_copy(acc.at[pl.ds(base, RPT)], out.at[cid, pl.ds(base, RPT)])


_R = 2560
_GRID = NP // _R


def _row_spec():
    return pl.BlockSpec((_R, D), lambda i: (i, 0))


def _dis_spec():
    return pl.BlockSpec((_R, 1), lambda i: (i, 0))


def _full_spec(r, c):
    return pl.BlockSpec((r, c), lambda i: (0, 0))


def _tc_first_body(x_ref, w_ref, dis_ref, ht_ref):
    ht_ref[...] = dis_ref[...] * jnp.dot(
        x_ref[...], w_ref[...], preferred_element_type=jnp.float32)


def _tc_first(xp, W0, dis):
    return pl.pallas_call(
        _tc_first_body,
        grid=(_GRID,),
        in_specs=[_row_spec(), _full_spec(D, D), _dis_spec()],
        out_specs=_row_spec(),
        out_shape=jax.ShapeDtypeStruct((NP, D), jnp.float32),
    )(xp, W0, dis)


def _tc_mid_body(s0_ref, s1_ref, ht_ref, dis_ref, b_ref, w_ref, h_ref, htn_ref):
    pre = dis_ref[...] * (s0_ref[...] + s1_ref[...] + ht_ref[...]) + b_ref[...]
    h = jnp.maximum(pre, 0.0)
    h_ref[...] = h
    htn_ref[...] = dis_ref[...] * jnp.dot(
        h, w_ref[...], preferred_element_type=jnp.float32)


def _tc_mid(s0, s1, ht, dis, b, Wn):
    return pl.pallas_call(
        _tc_mid_body,
        grid=(_GRID,),
        in_specs=[_row_spec(), _row_spec(), _row_spec(), _dis_spec(),
                  _full_spec(1, D), _full_spec(D, D)],
        out_specs=(_row_spec(), _row_spec()),
        out_shape=(jax.ShapeDtypeStruct((NP, D), jnp.float32),
                   jax.ShapeDtypeStruct((NP, D), jnp.float32)),
    )(s0, s1, ht, dis, b, Wn)


def _tc_last_body(s0_ref, s1_ref, ht_ref, dis_ref, b_ref, h0_ref, h1_ref,
                  wb_ref, out_ref):
    h2 = dis_ref[...] * (s0_ref[...] + s1_ref[...] + ht_ref[...]) + b_ref[...]
    out_ref[...] = (wb_ref[0:1, :] * h0_ref[...] + wb_ref[1:2, :] * h1_ref[...]
                    + wb_ref[2:3, :] * h2)


def _tc_last(s0, s1, ht, dis, b, h0, h1, wb):
    return pl.pallas_call(
        _tc_last_body,
        grid=(_GRID,),
        in_specs=[_row_spec(), _row_spec(), _row_spec(), _dis_spec(),
                  _full_spec(1, D), _row_spec(), _row_spec(), _full_spec(3, D)],
        out_specs=_row_spec(),
        out_shape=jax.ShapeDtypeStruct((NP, D), jnp.float32),
    )(s0, s1, ht, dis, b, h0, h1, wb)


def kernel(x, edge_index, W0, b0, W1, b1, W2, b2, jk_w):
    pad = EP - E
    padidx = jnp.int32(N) + (jnp.arange(pad, dtype=jnp.int32) % (NP - N))
    row3 = jnp.concatenate([edge_index[0], padidx]).reshape(NW, NCH, CH)
    col3 = jnp.concatenate([edge_index[1], padidx]).reshape(NW, NCH, CH)

    counts = _sc_hist(col3)
    cnt = counts[0] + counts[1]
    valid = jnp.arange(NP) < N
    dis = jnp.where(valid, lax.rsqrt(cnt + 1.0), 0.0).astype(jnp.float32)
    dis = dis.reshape(NP, 1)

    xp = jnp.concatenate([x, jnp.zeros((NP - N, D), jnp.float32)], axis=0)

    ht0 = _tc_first(xp, W0, dis)
    s0 = _sc_scatter(ht0, row3, col3)
    h0, ht1 = _tc_mid(s0[0], s0[1], ht0, dis, b0.reshape(1, D), W1)
    s1 = _sc_scatter(ht1, row3, col3)
    h1, ht2 = _tc_mid(s1[0], s1[1], ht1, dis, b1.reshape(1, D), W2)
    s2 = _sc_scatter(ht2, row3, col3)

    w = jax.nn.softmax(jk_w)
    wb = jnp.broadcast_to(w.reshape(3, 1), (3, D)).astype(jnp.float32)
    out = _tc_last(s2[0], s2[1], ht2, dis, b2.reshape(1, D), h0, h1, wb)
    return out[:N]


# speedup vs baseline: 1.0713x; 1.0011x over previous
import functools

import jax
import jax.numpy as jnp
from jax import lax
from jax.experimental import pallas as pl
from jax.experimental.pallas import tpu as pltpu
from jax.experimental.pallas import tpu_sc as plsc

N = 10000
NP = 10240
D = 128
E = 320000
EP = 327680
NW = 32
CH = 128
NCH = EP // (NW * CH)
RPT = NP // 16
_U = 8


def _mesh():
    return plsc.VectorSubcoreMesh(core_axis_name="c", subcore_axis_name="s")


@functools.partial(
    pl.kernel,
    mesh=_mesh(),
    out_type=jax.ShapeDtypeStruct((2, NP), jnp.float32),
    scratch_types=[
        pltpu.VMEM((NCH, CH), jnp.int32),
        pltpu.VMEM((CH,), jnp.float32),
        pltpu.VMEM((RPT,), jnp.float32),
        pltpu.VMEM_SHARED((NP,), jnp.float32),
    ],
)
def _sc_hist(col3, out, colbuf, ones_v, hbuf, hist):
    cid = lax.axis_index("c")
    sid = lax.axis_index("s")
    w = cid * 16 + sid
    for l in range(CH // 16):
        ones_v[pl.ds(l * 16, 16)] = jnp.ones((16,), jnp.float32)
    for l in range(RPT // 16):
        hbuf[pl.ds(l * 16, 16)] = jnp.zeros((16,), jnp.float32)
    pltpu.sync_copy(hbuf, hist.at[pl.ds(sid * RPT, RPT)])
    pltpu.sync_copy(col3.at[w], colbuf)
    plsc.subcore_barrier()

    def body(j, carry):
        pltpu.sync_copy(ones_v, hist.at[colbuf.at[j]], add=True)
        return carry

    lax.fori_loop(0, NCH, body, 0)
    plsc.subcore_barrier()
    pltpu.sync_copy(hist.at[pl.ds(sid * RPT, RPT)], hbuf)
    pltpu.sync_copy(hbuf, out.at[cid, pl.ds(sid * RPT, RPT)])


@functools.partial(
    pl.kernel,
    mesh=_mesh(),
    out_type=jax.ShapeDtypeStruct((2, NP, D), jnp.float32),
    scratch_types=[
        pltpu.VMEM((NCH, CH), jnp.int32),
        pltpu.VMEM((NCH, CH), jnp.int32),
        pltpu.VMEM((CH, D), jnp.float32),
        pltpu.VMEM((CH, D), jnp.float32),
        pltpu.VMEM((64, D), jnp.float32),
        pltpu.VMEM_SHARED((NP, D), jnp.float32),
        pltpu.SemaphoreType.DMA,
        pltpu.SemaphoreType.DMA,
    ],
)
def _sc_scatter(ht, row3, col3, out, rowbuf, colbuf, rv0, rv1, zbuf, acc,
                sem0, sem1):
    cid = lax.axis_index("c")
    sid = lax.axis_index("s")
    w = cid * 16 + sid
    for i in range(64):
        for l in range(D // 16):
            zbuf[i, pl.ds(l * 16, 16)] = jnp.zeros((16,), jnp.float32)
    base = sid * RPT
    for t in range(RPT // 64):
        pltpu.sync_copy(zbuf, acc.at[pl.ds(base + t * 64, 64)])
    pltpu.sync_copy(row3.at[w], rowbuf)
    pltpu.sync_copy(col3.at[w], colbuf)
    plsc.subcore_barrier()

    def body(j, carry):
        pltpu.async_copy(ht.at[rowbuf.at[j]], rv0, sem0).wait()
        pltpu.async_copy(rv0, acc.at[colbuf.at[j]], sem1, add=True).wait()
        return carry

    lax.fori_loop(0, NCH, body, 0)
    plsc.subcore_barrier()
    pltpu.sync_copy(acc.at[pl.ds(base, RPT)], out.at[cid, pl.ds(base, RPT)])


_R = 2560
_GRID = NP // _R


def _row_spec():
    return pl.BlockSpec((_R, D), lambda i: (i, 0))


def _dis_spec():
    return pl.BlockSpec((_R, 1), lambda i: (i, 0))


def _full_spec(r, c):
    return pl.BlockSpec((r, c), lambda i: (0, 0))


def _tc_matmul_body(x_ref, w_ref, xw_ref):
    xw_ref[...] = jnp.dot(
        x_ref[...], w_ref[...], preferred_element_type=jnp.float32)


def _tc_matmul(xp, W0):
    return pl.pallas_call(
        _tc_matmul_body,
        grid=(_GRID,),
        in_specs=[_row_spec(), _full_spec(D, D)],
        out_specs=_row_spec(),
        out_shape=jax.ShapeDtypeStruct((NP, D), jnp.float32),
    )(xp, W0)


def _tc_scale_body(xw_ref, dis_ref, ht_ref):
    ht_ref[...] = dis_ref[...] * xw_ref[...]


def _tc_scale(xw, dis):
    return pl.pallas_call(
        _tc_scale_body,
        grid=(_GRID,),
        in_specs=[_row_spec(), _dis_spec()],
        out_specs=_row_spec(),
        out_shape=jax.ShapeDtypeStruct((NP, D), jnp.float32),
    )(xw, dis)


def _tc_mid_body(s0_ref, s1_ref, ht_ref, dis_ref, b_ref, w_ref, h_ref, htn_ref):
    pre = dis_ref[...] * (s0_ref[...] + s1_ref[...] + ht_ref[...]) + b_ref[...]
    h = jnp.maximum(pre, 0.0)
    h_ref[...] = h
    htn_ref[...] = dis_ref[...] * jnp.dot(
        h, w_ref[...], preferred_element_type=jnp.float32)


def _tc_mid(s0, s1, ht, dis, b, Wn):
    return pl.pallas_call(
        _tc_mid_body,
        grid=(_GRID,),
        in_specs=[_row_spec(), _row_spec(), _row_spec(), _dis_spec(),
                  _full_spec(1, D), _full_spec(D, D)],
        out_specs=(_row_spec(), _row_spec()),
        out_shape=(jax.ShapeDtypeStruct((NP, D), jnp.float32),
                   jax.ShapeDtypeStruct((NP, D), jnp.float32)),
    )(s0, s1, ht, dis, b, Wn)


def _tc_last_body(s0_ref, s1_ref, ht_ref, dis_ref, b_ref, h0_ref, h1_ref,
                  wb_ref, out_ref):
    h2 = dis_ref[...] * (s0_ref[...] + s1_ref[...] + ht_ref[...]) + b_ref[...]
    out_ref[...] = (wb_ref[0:1, :] * h0_ref[...] + wb_ref[1:2, :] * h1_ref[...]
                    + wb_ref[2:3, :] * h2)


def _tc_last(s0, s1, ht, dis, b, h0, h1, wb):
    return pl.pallas_call(
        _tc_last_body,
        grid=(_GRID,),
        in_specs=[_row_spec(), _row_spec(), _row_spec(), _dis_spec(),
                  _full_spec(1, D), _row_spec(), _row_spec(), _full_spec(3, D)],
        out_specs=_row_spec(),
        out_shape=jax.ShapeDtypeStruct((NP, D), jnp.float32),
    )(s0, s1, ht, dis, b, h0, h1, wb)


def kernel(x, edge_index, W0, b0, W1, b1, W2, b2, jk_w):
    pad = EP - E
    padidx = jnp.int32(N) + (jnp.arange(pad, dtype=jnp.int32) % (NP - N))
    row3 = jnp.concatenate([edge_index[0], padidx]).reshape(NW, NCH, CH)
    col3 = jnp.concatenate([edge_index[1], padidx]).reshape(NW, NCH, CH)

    xp = jnp.concatenate([x, jnp.zeros((NP - N, D), jnp.float32)], axis=0)
    xw0 = _tc_matmul(xp, W0)

    counts = _sc_hist(col3)
    cnt = counts[0] + counts[1]
    valid = jnp.arange(NP) < N
    dis = jnp.where(valid, lax.rsqrt(cnt + 1.0), 0.0).astype(jnp.float32)
    dis = dis.reshape(NP, 1)

    ht0 = _tc_scale(xw0, dis)
    s0 = _sc_scatter(ht0, row3, col3)
    h0, ht1 = _tc_mid(s0[0], s0[1], ht0, dis, b0.reshape(1, D), W1)
    s1 = _sc_scatter(ht1, row3, col3)
    h1, ht2 = _tc_mid(s1[0], s1[1], ht1, dis, b1.reshape(1, D), W2)
    s2 = _sc_scatter(ht2, row3, col3)

    w = jax.nn.softmax(jk_w)
    wb = jnp.broadcast_to(w.reshape(3, 1), (3, D)).astype(jnp.float32)
    out = _tc_last(s2[0], s2[1], ht2, dis, b2.reshape(1, D), h0, h1, wb)
    return out[:N]


# speedup vs baseline: 1.0809x; 1.0090x over previous
import functools

import jax
import jax.numpy as jnp
from jax import lax
from jax.experimental import pallas as pl
from jax.experimental.pallas import tpu as pltpu
from jax.experimental.pallas import tpu_sc as plsc

N = 10000
NP = 10240
D = 128
E = 320000
NW = 32
EW = E // NW
CH = 128
NCH = EW // CH
TL = EW - NCH * CH
RPT = NP // 16
_U = 8


def _mesh():
    return plsc.VectorSubcoreMesh(core_axis_name="c", subcore_axis_name="s")


@functools.partial(
    pl.kernel,
    mesh=_mesh(),
    out_type=jax.ShapeDtypeStruct((2, NP), jnp.float32),
    scratch_types=[
        pltpu.VMEM((NCH, CH), jnp.int32),
        pltpu.VMEM((TL,), jnp.int32),
        pltpu.VMEM((CH,), jnp.float32),
        pltpu.VMEM((RPT,), jnp.float32),
        pltpu.VMEM_SHARED((NP,), jnp.float32),
    ],
)
def _sc_hist(col3, colt, out, colbuf, colbuf_t, ones_v, hbuf, hist):
    cid = lax.axis_index("c")
    sid = lax.axis_index("s")
    w = cid * 16 + sid
    for l in range(CH // 16):
        ones_v[pl.ds(l * 16, 16)] = jnp.ones((16,), jnp.float32)
    for l in range(RPT // 16):
        hbuf[pl.ds(l * 16, 16)] = jnp.zeros((16,), jnp.float32)
    pltpu.sync_copy(hbuf, hist.at[pl.ds(sid * RPT, RPT)])
    pltpu.sync_copy(col3.at[w], colbuf)
    pltpu.sync_copy(colt.at[w], colbuf_t)
    plsc.subcore_barrier()

    def body(j, carry):
        pltpu.sync_copy(ones_v, hist.at[colbuf.at[j]], add=True)
        return carry

    lax.fori_loop(0, NCH, body, 0)
    pltpu.sync_copy(ones_v.at[pl.ds(0, TL)], hist.at[colbuf_t], add=True)
    plsc.subcore_barrier()
    pltpu.sync_copy(hist.at[pl.ds(sid * RPT, RPT)], hbuf)
    pltpu.sync_copy(hbuf, out.at[cid, pl.ds(sid * RPT, RPT)])


@functools.partial(
    pl.kernel,
    mesh=_mesh(),
    out_type=jax.ShapeDtypeStruct((2, NP, D), jnp.float32),
    scratch_types=[
        pltpu.VMEM((NCH, CH), jnp.int32),
        pltpu.VMEM((NCH, CH), jnp.int32),
        pltpu.VMEM((TL,), jnp.int32),
        pltpu.VMEM((TL,), jnp.int32),
        pltpu.VMEM((CH, D), jnp.float32),
        pltpu.VMEM((TL, D), jnp.float32),
        pltpu.VMEM((64, D), jnp.float32),
        pltpu.VMEM_SHARED((NP, D), jnp.float32),
        pltpu.SemaphoreType.DMA,
        pltpu.SemaphoreType.DMA,
    ],
)
def _sc_scatter(ht, row3, col3, rowt, colt, out, rowbuf, colbuf, rowbuf_t,
                colbuf_t, rv0, rvt, zbuf, acc, sem0, sem1):
    cid = lax.axis_index("c")
    sid = lax.axis_index("s")
    w = cid * 16 + sid
    for i in range(64):
        for l in range(D // 16):
            zbuf[i, pl.ds(l * 16, 16)] = jnp.zeros((16,), jnp.float32)
    base = sid * RPT
    for t in range(RPT // 64):
        pltpu.sync_copy(zbuf, acc.at[pl.ds(base + t * 64, 64)])
    pltpu.sync_copy(row3.at[w], rowbuf)
    pltpu.sync_copy(col3.at[w], colbuf)
    pltpu.sync_copy(rowt.at[w], rowbuf_t)
    pltpu.sync_copy(colt.at[w], colbuf_t)
    plsc.subcore_barrier()

    def body(j, carry):
        pltpu.async_copy(ht.at[rowbuf.at[j]], rv0, sem0).wait()
        pltpu.async_copy(rv0, acc.at[colbuf.at[j]], sem1, add=True).wait()
        return carry

    lax.fori_loop(0, NCH, body, 0)
    pltpu.async_copy(ht.at[rowbuf_t], rvt, sem0).wait()
    pltpu.async_copy(rvt, acc.at[colbuf_t], sem1, add=True).wait()
    plsc.subcore_barrier()
    pltpu.sync_copy(acc.at[pl.ds(base, RPT)], out.at[cid, pl.ds(base, RPT)])


_R = 2560
_GRID = NP // _R


def _row_spec():
    return pl.BlockSpec((_R, D), lambda i: (i, 0))


def _dis_spec():
    return pl.BlockSpec((_R, 1), lambda i: (i, 0))


def _full_spec(r, c):
    return pl.BlockSpec((r, c), lambda i: (0, 0))


def _tc_matmul_body(x_ref, w_ref, xw_ref):
    xw_ref[...] = jnp.dot(
        x_ref[...], w_ref[...], preferred_element_type=jnp.float32)


def _tc_matmul(xp, W0):
    return pl.pallas_call(
        _tc_matmul_body,
        grid=(_GRID,),
        in_specs=[_row_spec(), _full_spec(D, D)],
        out_specs=_row_spec(),
        out_shape=jax.ShapeDtypeStruct((NP, D), jnp.float32),
    )(xp, W0)


def _tc_scale_body(xw_ref, dis_ref, ht_ref):
    ht_ref[...] = dis_ref[...] * xw_ref[...]


def _tc_scale(xw, dis):
    return pl.pallas_call(
        _tc_scale_body,
        grid=(_GRID,),
        in_specs=[_row_spec(), _dis_spec()],
        out_specs=_row_spec(),
        out_shape=jax.ShapeDtypeStruct((NP, D), jnp.float32),
    )(xw, dis)


def _tc_mid_body(s0_ref, s1_ref, ht_ref, dis_ref, b_ref, w_ref, h_ref, htn_ref):
    pre = dis_ref[...] * (s0_ref[...] + s1_ref[...] + ht_ref[...]) + b_ref[...]
    h = jnp.maximum(pre, 0.0)
    h_ref[...] = h
    htn_ref[...] = dis_ref[...] * jnp.dot(
        h, w_ref[...], preferred_element_type=jnp.float32)


def _tc_mid(s0, s1, ht, dis, b, Wn):
    return pl.pallas_call(
        _tc_mid_body,
        grid=(_GRID,),
        in_specs=[_row_spec(), _row_spec(), _row_spec(), _dis_spec(),
                  _full_spec(1, D), _full_spec(D, D)],
        out_specs=(_row_spec(), _row_spec()),
        out_shape=(jax.ShapeDtypeStruct((NP, D), jnp.float32),
                   jax.ShapeDtypeStruct((NP, D), jnp.float32)),
    )(s0, s1, ht, dis, b, Wn)


def _tc_last_body(s0_ref, s1_ref, ht_ref, dis_ref, b_ref, h0_ref, h1_ref,
                  wb_ref, out_ref):
    h2 = dis_ref[...] * (s0_ref[...] + s1_ref[...] + ht_ref[...]) + b_ref[...]
    out_ref[...] = (wb_ref[0:1, :] * h0_ref[...] + wb_ref[1:2, :] * h1_ref[...]
                    + wb_ref[2:3, :] * h2)


def _tc_last(s0, s1, ht, dis, b, h0, h1, wb):
    return pl.pallas_call(
        _tc_last_body,
        grid=(_GRID,),
        in_specs=[_row_spec(), _row_spec(), _row_spec(), _dis_spec(),
                  _full_spec(1, D), _row_spec(), _row_spec(), _full_spec(3, D)],
        out_specs=_row_spec(),
        out_shape=jax.ShapeDtypeStruct((NP, D), jnp.float32),
    )(s0, s1, ht, dis, b, h0, h1, wb)


def kernel(x, edge_index, W0, b0, W1, b1, W2, b2, jk_w):
    er = edge_index[0].reshape(NW, EW)
    ec = edge_index[1].reshape(NW, EW)
    row3 = er[:, :NCH * CH].reshape(NW, NCH, CH)
    col3 = ec[:, :NCH * CH].reshape(NW, NCH, CH)
    rowt = er[:, NCH * CH:]
    colt = ec[:, NCH * CH:]

    xp = jnp.concatenate([x, jnp.zeros((NP - N, D), jnp.float32)], axis=0)
    xw0 = _tc_matmul(xp, W0)

    counts = _sc_hist(col3, colt)
    cnt = counts[0] + counts[1]
    valid = jnp.arange(NP) < N
    dis = jnp.where(valid, lax.rsqrt(cnt + 1.0), 0.0).astype(jnp.float32)
    dis = dis.reshape(NP, 1)

    ht0 = _tc_scale(xw0, dis)
    s0 = _sc_scatter(ht0, row3, col3, rowt, colt)
    h0, ht1 = _tc_mid(s0[0], s0[1], ht0, dis, b0.reshape(1, D), W1)
    s1 = _sc_scatter(ht1, row3, col3, rowt, colt)
    h1, ht2 = _tc_mid(s1[0], s1[1], ht1, dis, b1.reshape(1, D), W2)
    s2 = _sc_scatter(ht2, row3, col3, rowt, colt)

    w = jax.nn.softmax(jk_w)
    wb = jnp.broadcast_to(w.reshape(3, 1), (3, D)).astype(jnp.float32)
    out = _tc_last(s2[0], s2[1], ht2, dis, b2.reshape(1, D), h0, h1, wb)
    return out[:N]


# speedup vs baseline: 1.0809x; 1.0000x over previous
import functools

import jax
import jax.numpy as jnp
from jax import lax
from jax.experimental import pallas as pl
from jax.experimental.pallas import tpu as pltpu
from jax.experimental.pallas import tpu_sc as plsc

N = 10000
NP = 10240
D = 128
E = 320000
NW = 32
EW = E // NW
CH = 128
NCH = EW // CH
TL = EW - NCH * CH
RPT = NP // 16


def _mesh():
    return plsc.VectorSubcoreMesh(core_axis_name="c", subcore_axis_name="s")


@functools.partial(
    pl.kernel,
    mesh=_mesh(),
    out_type=jax.ShapeDtypeStruct((2, NP), jnp.float32),
    scratch_types=[
        pltpu.VMEM((NCH, CH), jnp.int32),
        pltpu.VMEM((TL,), jnp.int32),
        pltpu.VMEM((CH,), jnp.float32),
        pltpu.VMEM((RPT,), jnp.float32),
        pltpu.VMEM_SHARED((NP,), jnp.float32),
    ],
)
def _sc_hist(col3, colt, out, colbuf, colbuf_t, ones_v, hbuf, hist):
    cid = lax.axis_index("c")
    sid = lax.axis_index("s")
    w = cid * 16 + sid
    for l in range(CH // 16):
        ones_v[pl.ds(l * 16, 16)] = jnp.ones((16,), jnp.float32)
    for l in range(RPT // 16):
        hbuf[pl.ds(l * 16, 16)] = jnp.zeros((16,), jnp.float32)
    pltpu.sync_copy(hbuf, hist.at[pl.ds(sid * RPT, RPT)])
    pltpu.sync_copy(col3.at[w], colbuf)
    pltpu.sync_copy(colt.at[w], colbuf_t)
    plsc.subcore_barrier()

    def body(j, carry):
        pltpu.sync_copy(ones_v, hist.at[colbuf.at[j]], add=True)
        return carry

    lax.fori_loop(0, NCH, body, 0)
    pltpu.sync_copy(ones_v.at[pl.ds(0, TL)], hist.at[colbuf_t], add=True)
    plsc.subcore_barrier()
    pltpu.sync_copy(hist.at[pl.ds(sid * RPT, RPT)], hbuf)
    pltpu.sync_copy(hbuf, out.at[cid, pl.ds(sid * RPT, RPT)])


@functools.partial(
    pl.kernel,
    mesh=_mesh(),
    out_type=jax.ShapeDtypeStruct((2, NP, D), jnp.float32),
    scratch_types=[
        pltpu.VMEM((NCH, CH), jnp.int32),
        pltpu.VMEM((NCH, CH), jnp.int32),
        pltpu.VMEM((TL,), jnp.int32),
        pltpu.VMEM((TL,), jnp.int32),
        pltpu.VMEM((CH, D), jnp.float32),
        pltpu.VMEM((TL, D), jnp.float32),
        pltpu.VMEM((64, D), jnp.float32),
        pltpu.VMEM_SHARED((NP, D), jnp.float32),
        pltpu.SemaphoreType.DMA,
        pltpu.SemaphoreType.DMA,
    ],
)
def _sc_scatter(ht, row3, col3, rowt, colt, out, rowbuf, colbuf, rowbuf_t,
                colbuf_t, rv0, rvt, zbuf, acc, sem0, sem1):
    cid = lax.axis_index("c")
    sid = lax.axis_index("s")
    w = cid * 16 + sid
    for i in range(64):
        for l in range(D // 16):
            zbuf[i, pl.ds(l * 16, 16)] = jnp.zeros((16,), jnp.float32)
    base = sid * RPT
    for t in range(RPT // 64):
        pltpu.sync_copy(zbuf, acc.at[pl.ds(base + t * 64, 64)])
    pltpu.sync_copy(row3.at[w], rowbuf)
    pltpu.sync_copy(col3.at[w], colbuf)
    pltpu.sync_copy(rowt.at[w], rowbuf_t)
    pltpu.sync_copy(colt.at[w], colbuf_t)
    plsc.subcore_barrier()

    def body(j, carry):
        pltpu.async_copy(ht.at[rowbuf.at[j]], rv0, sem0).wait()
        pltpu.async_copy(rv0, acc.at[colbuf.at[j]], sem1, add=True).wait()
        return carry

    lax.fori_loop(0, NCH, body, 0)
    pltpu.async_copy(ht.at[rowbuf_t], rvt, sem0).wait()
    pltpu.async_copy(rvt, acc.at[colbuf_t], sem1, add=True).wait()
    plsc.subcore_barrier()
    pltpu.sync_copy(acc.at[pl.ds(base, RPT)], out.at[cid, pl.ds(base, RPT)])


_R = 2560
_GRID = NP // _R


def _row_spec():
    return pl.BlockSpec((_R, D), lambda i: (i, 0))


def _dis_spec():
    return pl.BlockSpec((_R, 1), lambda i: (i, 0))


def _full_spec(r, c):
    return pl.BlockSpec((r, c), lambda i: (0, 0))


def _tc_matmul_body(x_ref, w_ref, xw_ref):
    xw_ref[...] = jnp.dot(
        x_ref[...], w_ref[...], preferred_element_type=jnp.float32)


def _tc_matmul(xp, W0):
    return pl.pallas_call(
        _tc_matmul_body,
        grid=(_GRID,),
        in_specs=[_row_spec(), _full_spec(D, D)],
        out_specs=_row_spec(),
        out_shape=jax.ShapeDtypeStruct((NP, D), jnp.float32),
    )(xp, W0)


def _tc_scale_body(xw_ref, dis_ref, ht_ref):
    ht_ref[...] = dis_ref[...] * xw_ref[...]


def _tc_scale(xw, dis):
    return pl.pallas_call(
        _tc_scale_body,
        grid=(_GRID,),
        in_specs=[_row_spec(), _dis_spec()],
        out_specs=_row_spec(),
        out_shape=jax.ShapeDtypeStruct((NP, D), jnp.float32),
    )(xw, dis)


def _tc_mid_body(s0_ref, s1_ref, ht_ref, dis_ref, b_ref, w_ref, h_ref, htn_ref):
    pre = dis_ref[...] * (s0_ref[...] + s1_ref[...] + ht_ref[...]) + b_ref[...]
    h = jnp.maximum(pre, 0.0)
    h_ref[...] = h
    htn_ref[...] = dis_ref[...] * jnp.dot(
        h, w_ref[...], preferred_element_type=jnp.float32)


def _tc_mid(s0, s1, ht, dis, b, Wn):
    return pl.pallas_call(
        _tc_mid_body,
        grid=(_GRID,),
        in_specs=[_row_spec(), _row_spec(), _row_spec(), _dis_spec(),
                  _full_spec(1, D), _full_spec(D, D)],
        out_specs=(_row_spec(), _row_spec()),
        out_shape=(jax.ShapeDtypeStruct((NP, D), jnp.float32),
                   jax.ShapeDtypeStruct((NP, D), jnp.float32)),
    )(s0, s1, ht, dis, b, Wn)


def _tc_last_body(s0_ref, s1_ref, ht_ref, dis_ref, b_ref, h0_ref, h1_ref,
                  wb_ref, out_ref):
    h2 = dis_ref[...] * (s0_ref[...] + s1_ref[...] + ht_ref[...]) + b_ref[...]
    out_ref[...] = (wb_ref[0:1, :] * h0_ref[...] + wb_ref[1:2, :] * h1_ref[...]
                    + wb_ref[2:3, :] * h2)


def _tc_last(s0, s1, ht, dis, b, h0, h1, wb):
    return pl.pallas_call(
        _tc_last_body,
        grid=(_GRID,),
        in_specs=[_row_spec(), _row_spec(), _row_spec(), _dis_spec(),
                  _full_spec(1, D), _row_spec(), _row_spec(), _full_spec(3, D)],
        out_specs=_row_spec(),
        out_shape=jax.ShapeDtypeStruct((NP, D), jnp.float32),
    )(s0, s1, ht, dis, b, h0, h1, wb)


def kernel(x, edge_index, W0, b0, W1, b1, W2, b2, jk_w):
    er = edge_index[0].reshape(NW, EW)
    ec = edge_index[1].reshape(NW, EW)
    row3 = er[:, :NCH * CH].reshape(NW, NCH, CH)
    col3 = ec[:, :NCH * CH].reshape(NW, NCH, CH)
    rowt = er[:, NCH * CH:]
    colt = ec[:, NCH * CH:]

    xp = jnp.concatenate([x, jnp.zeros((NP - N, D), jnp.float32)], axis=0)
    xw0 = _tc_matmul(xp, W0)

    counts = _sc_hist(col3, colt)
    cnt = counts[0] + counts[1]
    valid = jnp.arange(NP) < N
    dis = jnp.where(valid, lax.rsqrt(cnt + 1.0), 0.0).astype(jnp.float32)
    dis = dis.reshape(NP, 1)

    ht0 = _tc_scale(xw0, dis)
    s0 = _sc_scatter(ht0, row3, col3, rowt, colt)
    h0, ht1 = _tc_mid(s0[0], s0[1], ht0, dis, b0.reshape(1, D), W1)
    s1 = _sc_scatter(ht1, row3, col3, rowt, colt)
    h1, ht2 = _tc_mid(s1[0], s1[1], ht1, dis, b1.reshape(1, D), W2)
    s2 = _sc_scatter(ht2, row3, col3, rowt, colt)

    w = jax.nn.softmax(jk_w)
    wb = jnp.broadcast_to(w.reshape(3, 1), (3, D)).astype(jnp.float32)
    out = _tc_last(s2[0], s2[1], ht2, dis, b2.reshape(1, D), h0, h1, wb)
    return out[:N]
